# Initial kernel scaffold; baseline (speedup 1.0000x reference)
import jax
import jax.numpy as jnp
from jax.experimental import pallas as pl


def kernel(hidden_states, ln1_w, qkv_w, qkv_b, o_w, ln2_w, gate_w, expert_gate_up, expert_down, shared_gate_up, shared_down, shared_gate_vec, positions):
    raise NotImplementedError("write your pallas kernel here")


# speedup vs baseline: 1.1681x; 1.1681x over previous
import functools

import jax
import jax.numpy as jnp
from jax.experimental import pallas as pl

B, S, D, H, DH = 1, 2048, 768, 12, 64
E, TOPK, DFF, SDFF = 8, 2, 512, 2048
EPS = 1e-6
THETA = 10000.0

RB = 256
NRB = S // RB

_bf = jnp.bfloat16
_f32 = jnp.float32


def _dot(a, b):
    return jax.lax.dot_general(a, b, (((1,), (0,)), ((), ())),
                               preferred_element_type=_f32)


def _pre_attn_kernel(x_ref, ln1_ref, wqkv_ref, b_ref, q_ref, k_ref, v_ref):
    i = pl.program_id(0)
    x = x_ref[...]
    var = jnp.mean(x * x, axis=-1, keepdims=True)
    xn = x * jax.lax.rsqrt(var + EPS) * ln1_ref[...]
    qkv = _dot(xn.astype(_bf), wqkv_ref[...]) + b_ref[...]

    half = DH // 2
    freq = jax.lax.broadcasted_iota(jnp.int32, (1, half), 1).astype(_f32)
    inv = 1.0 / (THETA ** (freq / half))
    pos = (i * RB + jax.lax.broadcasted_iota(jnp.int32, (RB, 1), 0)).astype(_f32)
    ang = pos * inv
    cos = jnp.tile(jnp.cos(ang), (1, 2 * H))
    sin = jnp.tile(jnp.sin(ang), (1, 2 * H))
    col = jax.lax.broadcasted_iota(jnp.int32, (RB, H * DH), 1)
    first_half = (col % DH) < half

    def rope(t):
        t_rot = jnp.where(first_half,
                          -jnp.roll(t, -half, axis=1),
                          jnp.roll(t, half, axis=1))
        return t * cos + t_rot * sin

    q = rope(qkv[:, :H * DH]).astype(_bf)
    k = rope(qkv[:, H * DH:2 * H * DH]).astype(_bf)
    v = qkv[:, 2 * H * DH:].astype(_bf)
    for h in range(H):
        q_ref[h, :, :] = q[:, h * DH:(h + 1) * DH]
        k_ref[h, :, :] = k[:, h * DH:(h + 1) * DH]
        v_ref[h, :, :] = v[:, h * DH:(h + 1) * DH]


def _pre_attn(x, ln1_w, wqkv_bf, qkv_b):
    out_sh = jax.ShapeDtypeStruct((H, S, DH), _bf)
    return pl.pallas_call(
        _pre_attn_kernel,
        grid=(NRB,),
        in_specs=[
            pl.BlockSpec((RB, D), lambda i: (i, 0)),
            pl.BlockSpec((D,), lambda i: (0,)),
            pl.BlockSpec((D, 3 * H * DH), lambda i: (0, 0)),
            pl.BlockSpec((3 * H * DH,), lambda i: (0,)),
        ],
        out_specs=[
            pl.BlockSpec((H, RB, DH), lambda i: (0, i, 0)),
            pl.BlockSpec((H, RB, DH), lambda i: (0, i, 0)),
            pl.BlockSpec((H, RB, DH), lambda i: (0, i, 0)),
        ],
        out_shape=[out_sh, out_sh, out_sh],
    )(x, ln1_w, wqkv_bf, qkv_b)


def _flash_kernel(q_ref, k_ref, v_ref, o_ref):
    qi = pl.program_id(1)
    q = q_ref[0]
    scale = DH ** -0.5

    m0 = jnp.full((RB, 1), -1e30, _f32)
    l0 = jnp.zeros((RB, 1), _f32)
    a0 = jnp.zeros((RB, DH), _f32)

    rows = qi * RB + jax.lax.broadcasted_iota(jnp.int32, (RB, RB), 0)

    def body(j, carry):
        m, l, acc = carry
        kj = k_ref[0, pl.ds(j * RB, RB), :]
        vj = v_ref[0, pl.ds(j * RB, RB), :]
        s = jax.lax.dot_general(q, kj, (((1,), (1,)), ((), ())),
                                preferred_element_type=_f32) * scale
        cols = j * RB + jax.lax.broadcasted_iota(jnp.int32, (RB, RB), 1)
        s = jnp.where(rows >= cols, s, -1e30)
        m_new = jnp.maximum(m, jnp.max(s, axis=1, keepdims=True))
        p = jnp.exp(s - m_new)
        alpha = jnp.exp(m - m_new)
        l_new = l * alpha + jnp.sum(p, axis=1, keepdims=True)
        acc_new = acc * alpha + _dot(p.astype(_bf), vj)
        return m_new, l_new, acc_new

    m, l, acc = jax.lax.fori_loop(0, qi + 1, body, (m0, l0, a0))
    o_ref[0] = (acc / l).astype(_f32)


def _attention(q, k, v):
    return pl.pallas_call(
        _flash_kernel,
        grid=(H, NRB),
        in_specs=[
            pl.BlockSpec((1, RB, DH), lambda h, i: (h, i, 0)),
            pl.BlockSpec((1, S, DH), lambda h, i: (h, 0, 0)),
            pl.BlockSpec((1, S, DH), lambda h, i: (h, 0, 0)),
        ],
        out_specs=pl.BlockSpec((1, RB, DH), lambda h, i: (h, i, 0)),
        out_shape=jax.ShapeDtypeStruct((H, S, DH), _f32),
    )(q, k, v)


def _post_attn_kernel(attn_ref, hid_ref, ow_ref, ln2_ref, gw_ref, sgv_ref,
                      h2_ref, xnb_ref, comb_ref, sg_ref):
    attn = jnp.concatenate([attn_ref[h] for h in range(H)], axis=1)
    h2 = hid_ref[...] + _dot(attn.astype(_bf), ow_ref[...])
    h2_ref[...] = h2
    var = jnp.mean(h2 * h2, axis=-1, keepdims=True)
    xn = h2 * jax.lax.rsqrt(var + EPS) * ln2_ref[...]
    xnb_ref[...] = xn.astype(_bf)

    logits = jax.lax.dot_general(xn, gw_ref[...], (((1,), (0,)), ((), ())),
                                 preferred_element_type=_f32,
                                 precision=jax.lax.Precision.HIGHEST)
    p = jax.nn.softmax(logits, axis=-1)
    lane = jax.lax.broadcasted_iota(jnp.int32, (RB, E), 1)
    i1 = jnp.argmax(p, axis=-1, keepdims=True)
    m1 = jnp.max(p, axis=-1, keepdims=True)
    oh1 = lane == i1
    p2 = jnp.where(oh1, -1.0, p)
    i2 = jnp.argmax(p2, axis=-1, keepdims=True)
    m2 = jnp.max(p2, axis=-1, keepdims=True)
    oh2 = lane == i2
    denom = m1 + m2
    comb_ref[...] = jnp.where(oh1, m1 / denom,
                              jnp.where(oh2, m2 / denom, 0.0))

    sgl = jax.lax.dot_general(xn, sgv_ref[...], (((1,), (0,)), ((), ())),
                              preferred_element_type=_f32)
    sg_ref[...] = jax.nn.sigmoid(sgl)


def _post_attn(attn, hid, ow_bf, ln2_w, gate_wT, sgv):
    return pl.pallas_call(
        _post_attn_kernel,
        grid=(NRB,),
        in_specs=[
            pl.BlockSpec((H, RB, DH), lambda i: (0, i, 0)),
            pl.BlockSpec((RB, D), lambda i: (i, 0)),
            pl.BlockSpec((H * DH, D), lambda i: (0, 0)),
            pl.BlockSpec((D,), lambda i: (0,)),
            pl.BlockSpec((D, E), lambda i: (0, 0)),
            pl.BlockSpec((D, 1), lambda i: (0, 0)),
        ],
        out_specs=[
            pl.BlockSpec((RB, D), lambda i: (i, 0)),
            pl.BlockSpec((RB, D), lambda i: (i, 0)),
            pl.BlockSpec((RB, E), lambda i: (i, 0)),
            pl.BlockSpec((RB, 1), lambda i: (i, 0)),
        ],
        out_shape=[
            jax.ShapeDtypeStruct((S, D), _f32),
            jax.ShapeDtypeStruct((S, D), _bf),
            jax.ShapeDtypeStruct((S, E), _f32),
            jax.ShapeDtypeStruct((S, 1), _f32),
        ],
    )(attn, hid, ow_bf, ln2_w, gate_wT, sgv)


def _shared_kernel(xnb_ref, h2_ref, sg_ref, sgu_ref, sdown_ref, base_ref):
    xnb = xnb_ref[...]
    gu = _dot(xnb, sgu_ref[...])
    g = gu[:, :SDFF]
    u = gu[:, SDFF:]
    act = (g * jax.nn.sigmoid(g) * u).astype(_bf)
    sh = _dot(act, sdown_ref[...])
    base_ref[...] = h2_ref[...] + sg_ref[...] * sh


def _shared(xnb, h2, sg, sgu_bf, sdown_bf):
    return pl.pallas_call(
        _shared_kernel,
        grid=(NRB,),
        in_specs=[
            pl.BlockSpec((RB, D), lambda i: (i, 0)),
            pl.BlockSpec((RB, D), lambda i: (i, 0)),
            pl.BlockSpec((RB, 1), lambda i: (i, 0)),
            pl.BlockSpec((D, 2 * SDFF), lambda i: (0, 0)),
            pl.BlockSpec((SDFF, D), lambda i: (0, 0)),
        ],
        out_specs=pl.BlockSpec((RB, D), lambda i: (i, 0)),
        out_shape=jax.ShapeDtypeStruct((S, D), _f32),
    )(xnb, h2, sg, sgu_bf, sdown_bf)


def _moe_kernel(xnb_ref, comb_ref, egu_ref, edown_ref, base_ref, out_ref):
    e = pl.program_id(1)
    xnb = xnb_ref[...]
    gu = _dot(xnb, egu_ref[0])
    g = gu[:, :DFF]
    u = gu[:, DFF:]
    act = (g * jax.nn.sigmoid(g) * u).astype(_bf)
    oe = _dot(act, edown_ref[0])
    lane = jax.lax.broadcasted_iota(jnp.int32, (RB, E), 1)
    ce = jnp.sum(jnp.where(lane == e, comb_ref[...], 0.0),
                 axis=1, keepdims=True)

    @pl.when(e == 0)
    def _():
        out_ref[...] = base_ref[...] + ce * oe

    @pl.when(e != 0)
    def _():
        out_ref[...] += ce * oe


def _moe(xnb, comb, egu_bf, edown_bf, base):
    return pl.pallas_call(
        _moe_kernel,
        grid=(NRB, E),
        in_specs=[
            pl.BlockSpec((RB, D), lambda i, e: (i, 0)),
            pl.BlockSpec((RB, E), lambda i, e: (i, 0)),
            pl.BlockSpec((1, D, 2 * DFF), lambda i, e: (e, 0, 0)),
            pl.BlockSpec((1, DFF, D), lambda i, e: (e, 0, 0)),
            pl.BlockSpec((RB, D), lambda i, e: (i, 0)),
        ],
        out_specs=pl.BlockSpec((RB, D), lambda i, e: (i, 0)),
        out_shape=jax.ShapeDtypeStruct((S, D), _f32),
    )(xnb, comb, egu_bf, edown_bf, base)


def kernel(hidden_states, ln1_w, qkv_w, qkv_b, o_w, ln2_w, gate_w,
           expert_gate_up, expert_down, shared_gate_up, shared_down,
           shared_gate_vec, positions):
    x = hidden_states.reshape(S, D)
    wqkv_bf = qkv_w.T.astype(_bf)
    ow_bf = o_w.T.astype(_bf)
    gate_wT = gate_w.T
    sgu_bf = shared_gate_up.astype(_bf)
    sdown_bf = shared_down.astype(_bf)
    egu_bf = expert_gate_up.astype(_bf)
    edown_bf = expert_down.astype(_bf)

    q, k, v = _pre_attn(x, ln1_w, wqkv_bf, qkv_b)
    attn = _attention(q, k, v)
    h2, xnb, comb, sg = _post_attn(attn, x, ow_bf, ln2_w, gate_wT,
                                   shared_gate_vec)
    base = _shared(xnb, h2, sg, sgu_bf, sdown_bf)
    out = _moe(xnb, comb, egu_bf, edown_bf, base)
    return out.reshape(B, S, D)
